# Initial kernel scaffold; baseline (speedup 1.0000x reference)
import jax
import jax.numpy as jnp
from jax.experimental import pallas as pl


def kernel(x, edge_src, edge_dst, edge_emb, time_weight, message_node_scale, ln1_g, ln1_b, ln2_g, ln2_b, W_self, b_self, W_msg1, b_msg1, W_msg2, b_msg2, W_gate1, b_gate1, W_gate2, b_gate2, W_agg, b_agg, W_ffn1, b_ffn1, W_ffn2, b_ffn2):
    raise NotImplementedError("write your pallas kernel here")


# speedup vs baseline: 2.6358x; 2.6358x over previous
import functools

import jax
import jax.numpy as jnp
from jax import lax
from jax.experimental import pallas as pl
from jax.experimental.pallas import tpu as pltpu
from jax.experimental.pallas import tpu_sc as plsc

NW = 32
SB = 128
BE = 512
BN = 512
WS = 272
WD = 144

_INV_SQRT2 = 0.7071067811865476


def _gelu(z):
    return 0.5 * z * (1.0 + lax.erf(z * _INV_SQRT2))


def _ln_rows(xb, g, b, eps=1e-5):
    m = jnp.mean(xb, axis=1, keepdims=True)
    v = jnp.mean((xb - m) ** 2, axis=1, keepdims=True)
    return (xb - m) / jnp.sqrt(v + eps) * g + b


def _prep_body(x_ref, s_ref, g_ref, b_ref, W_ref, bc_ref, h_ref, ts_ref, td_ref):
    h = _ln_rows(x_ref[...], g_ref[...], b_ref[...])
    h_ref[...] = h
    P = jnp.dot(h, W_ref[...], preferred_element_type=jnp.float32) + bc_ref[...]
    sb = s_ref[...]
    pad15 = jnp.zeros((P.shape[0], 15), jnp.float32)
    ts_ref[...] = jnp.concatenate([P[:, :256], sb, pad15], axis=1)
    td_ref[...] = jnp.concatenate([P[:, 256:], sb, pad15], axis=1)


def _mlp_body(gs_ref, gd_ref, emb_ref, tw_ref, Wem_ref, Weg_ref, Wm2_ref,
              bm2_ref, wg2_ref, bg2_ref, out_ref):
    e = emb_ref[...]
    gs = gs_ref[...]
    gd = gd_ref[...]
    mpre = gs[:, :128] + jnp.dot(e, Wem_ref[...], preferred_element_type=jnp.float32)
    gpre = gd[:, :128] + gs[:, 128:256] + jnp.dot(
        e, Weg_ref[...], preferred_element_type=jnp.float32)
    msg = jnp.dot(_gelu(mpre), Wm2_ref[...],
                  preferred_element_type=jnp.float32) + bm2_ref[...]
    glog = jnp.sum(_gelu(gpre) * wg2_ref[...], axis=1, keepdims=True) + bg2_ref[0, 0]
    gate = jax.nn.sigmoid(glog)
    scale = 0.5 * tw_ref[...] * (gs[:, 256:257] + gd[:, 128:129])
    out_ref[...] = msg * gate * scale


def _final_body(x_ref, h_ref, ap_ref, deg_ref, Ws_ref, bs_ref, Wa_ref, ba_ref,
                g2_ref, b2_ref, Wf1_ref, bf1_ref, Wf2_ref, bf2_ref, out_ref):
    deg = jnp.sum(deg_ref[...], axis=0)
    agg = (ap_ref[0] + ap_ref[1]) / jnp.clip(deg, 1.0, None)[:, None]
    upd = (jnp.dot(h_ref[...], Ws_ref[...], preferred_element_type=jnp.float32)
           + bs_ref[...]
           + jnp.dot(agg, Wa_ref[...], preferred_element_type=jnp.float32)
           + ba_ref[...])
    o1 = x_ref[...] + upd
    f = _ln_rows(o1, g2_ref[...], b2_ref[...])
    ffn = jnp.dot(_gelu(jnp.dot(f, Wf1_ref[...], preferred_element_type=jnp.float32)
                        + bf1_ref[...]),
                  Wf2_ref[...], preferred_element_type=jnp.float32) + bf2_ref[...]
    out_ref[...] = o1 + ffn


def kernel(x, edge_src, edge_dst, edge_emb, time_weight, message_node_scale,
           ln1_g, ln1_b, ln2_g, ln2_b, W_self, b_self, W_msg1, b_msg1,
           W_msg2, b_msg2, W_gate1, b_gate1, W_gate2, b_gate2, W_agg, b_agg,
           W_ffn1, b_ffn1, W_ffn2, b_ffn2):
    N, D = x.shape
    E = edge_src.shape[0]
    DE = edge_emb.shape[1]
    f32 = jnp.float32

    NP = -(-N // 2048) * 2048
    EP = -(-E // (NW * SB)) * (NW * SB)
    per_w = EP // NW
    n_blk = per_w // SB
    n_stripe = NP // 16 // SB

    xp = jnp.pad(x, ((0, NP - N), (0, 0)))
    src_p = jnp.pad(edge_src.astype(jnp.int32), (0, EP - E))
    dst_p = jnp.pad(edge_dst.astype(jnp.int32), (0, EP - E))
    emb_p = jnp.pad(edge_emb, ((0, EP - E), (0, 0)))
    tw_p = jnp.pad(time_weight, ((0, EP - E), (0, 0)))
    valid = jnp.pad(jnp.ones((E,), f32), (0, EP - E))
    s_p = jnp.pad(message_node_scale, ((0, NP - N), (0, 0)))

    Wcat = jnp.concatenate([W_msg1[:D], W_gate1[D:2 * D], W_gate1[:D]], axis=1)
    bcat = jnp.concatenate([b_msg1, jnp.zeros((D,), f32), b_gate1])[None, :]
    Wem = W_msg1[D:]
    Weg = W_gate1[2 * D:]

    row1 = lambda a: a[None, :]

    h, t_src, t_dst = pl.pallas_call(
        _prep_body,
        grid=(NP // BN,),
        in_specs=[
            pl.BlockSpec((BN, D), lambda i: (i, 0)),
            pl.BlockSpec((BN, 1), lambda i: (i, 0)),
            pl.BlockSpec((1, D), lambda i: (0, 0)),
            pl.BlockSpec((1, D), lambda i: (0, 0)),
            pl.BlockSpec((D, 3 * D), lambda i: (0, 0)),
            pl.BlockSpec((1, 3 * D), lambda i: (0, 0)),
        ],
        out_specs=[
            pl.BlockSpec((BN, D), lambda i: (i, 0)),
            pl.BlockSpec((BN, WS), lambda i: (i, 0)),
            pl.BlockSpec((BN, WD), lambda i: (i, 0)),
        ],
        out_shape=[
            jax.ShapeDtypeStruct((NP, D), f32),
            jax.ShapeDtypeStruct((NP, WS), f32),
            jax.ShapeDtypeStruct((NP, WD), f32),
        ],
    )(xp, s_p, row1(ln1_g), row1(ln1_b), Wcat, bcat)

    mesh = plsc.VectorSubcoreMesh(core_axis_name="c", subcore_axis_name="s")

    @functools.partial(
        pl.kernel,
        out_type=[
            jax.ShapeDtypeStruct((EP, WS), f32),
            jax.ShapeDtypeStruct((EP, WD), f32),
        ],
        mesh=mesh,
        scratch_types=[
            pltpu.VMEM((SB,), jnp.int32),
            pltpu.VMEM((SB,), jnp.int32),
            pltpu.VMEM((SB, WS), f32),
            pltpu.VMEM((SB, WD), f32),
            pltpu.SemaphoreType.DMA,
            pltpu.SemaphoreType.DMA,
        ],
        compiler_params=pltpu.CompilerParams(use_tc_tiling_on_sc=False, needs_layout_passes=False),
    )
    def _gather(ts_hbm, td_hbm, src_hbm, dst_hbm, gsrc_hbm, gdst_hbm,
                isrc, idst, bufA, bufB, semA, semB):
        wid = lax.axis_index("s") * 2 + lax.axis_index("c")

        def body(g, carry):
            base = wid * per_w + g * SB
            pltpu.sync_copy(src_hbm.at[pl.ds(base, SB)], isrc)
            pltpu.sync_copy(dst_hbm.at[pl.ds(base, SB)], idst)
            cpA = pltpu.async_copy(ts_hbm.at[isrc], bufA, semA)
            cpB = pltpu.async_copy(td_hbm.at[idst], bufB, semB)
            cpA.wait()
            cpB.wait()
            pltpu.sync_copy(bufA, gsrc_hbm.at[pl.ds(base, SB)])
            pltpu.sync_copy(bufB, gdst_hbm.at[pl.ds(base, SB)])
            return carry

        lax.fori_loop(0, n_blk, body, 0)

    g_src, g_dst = _gather(t_src, t_dst, src_p, dst_p)

    edge_repr = pl.pallas_call(
        _mlp_body,
        grid=(EP // BE,),
        in_specs=[
            pl.BlockSpec((BE, WS), lambda i: (i, 0)),
            pl.BlockSpec((BE, WD), lambda i: (i, 0)),
            pl.BlockSpec((BE, DE), lambda i: (i, 0)),
            pl.BlockSpec((BE, 1), lambda i: (i, 0)),
            pl.BlockSpec((DE, D), lambda i: (0, 0)),
            pl.BlockSpec((DE, D), lambda i: (0, 0)),
            pl.BlockSpec((D, D), lambda i: (0, 0)),
            pl.BlockSpec((1, D), lambda i: (0, 0)),
            pl.BlockSpec((1, D), lambda i: (0, 0)),
            pl.BlockSpec((1, 1), lambda i: (0, 0)),
        ],
        out_specs=pl.BlockSpec((BE, D), lambda i: (i, 0)),
        out_shape=jax.ShapeDtypeStruct((EP, D), f32),
    )(g_src, g_dst, emb_p, tw_p, Wem, Weg,
      W_msg2, row1(b_msg2), W_gate2.reshape(1, D), b_gate2.reshape(1, 1))

    zeros_blk = jnp.zeros((SB, D), f32)

    @functools.partial(
        pl.kernel,
        out_type=[
            jax.ShapeDtypeStruct((2, NP, D), f32),
            jax.ShapeDtypeStruct((NW, NP), f32),
        ],
        mesh=mesh,
        scratch_types=[
            pltpu.VMEM((SB, D), f32),
            pltpu.VMEM((SB,), jnp.int32),
            pltpu.VMEM((SB,), f32),
            pltpu.VMEM((NP,), f32),
            pltpu.VMEM_SHARED((NP, D), f32),
        ],
        compiler_params=pltpu.CompilerParams(use_tc_tiling_on_sc=False, needs_layout_passes=False),
    )
    def _scatter(er_hbm, dst_hbm, valid_hbm, z_hbm, agg_hbm, deg_hbm,
                 rows, dstb, validb, deg_v, agg_sh):
        c = lax.axis_index("c")
        s = lax.axis_index("s")
        wid = s * 2 + c
        zero16 = jnp.zeros((16,), f32)

        def zdeg(i, carry):
            deg_v[pl.ds(i * 16, 16)] = zero16
            return carry
        lax.fori_loop(0, NP // 16, zdeg, 0)

        pltpu.sync_copy(z_hbm, rows)
        for j in range(n_stripe):
            pltpu.sync_copy(rows, agg_sh.at[pl.ds(s * (NP // 16) + j * SB, SB)])
        plsc.subcore_barrier()

        def body(g, carry):
            base = wid * per_w + g * SB
            pltpu.sync_copy(dst_hbm.at[pl.ds(base, SB)], dstb)
            pltpu.sync_copy(valid_hbm.at[pl.ds(base, SB)], validb)
            pltpu.sync_copy(er_hbm.at[pl.ds(base, SB)], rows)
            pltpu.sync_copy(rows, agg_sh.at[dstb], add=True)
            for i in range(SB // 16):
                sl = pl.ds(i * 16, 16)
                plsc.addupdate_scatter(deg_v, [dstb[sl]], validb[sl])
            return carry

        lax.fori_loop(0, n_blk, body, 0)
        plsc.subcore_barrier()

        for j in range(n_stripe):
            r0 = s * (NP // 16) + j * SB
            pltpu.sync_copy(agg_sh.at[pl.ds(r0, SB)], rows)
            pltpu.sync_copy(rows, agg_hbm.at[c].at[pl.ds(r0, SB)])
        pltpu.sync_copy(deg_v, deg_hbm.at[wid])

    agg_part, deg_part = _scatter(edge_repr, dst_p, valid, zeros_blk)

    out = pl.pallas_call(
        _final_body,
        grid=(NP // BN,),
        in_specs=[
            pl.BlockSpec((BN, D), lambda i: (i, 0)),
            pl.BlockSpec((BN, D), lambda i: (i, 0)),
            pl.BlockSpec((2, BN, D), lambda i: (0, i, 0)),
            pl.BlockSpec((NW, BN), lambda i: (0, i)),
            pl.BlockSpec((D, D), lambda i: (0, 0)),
            pl.BlockSpec((1, D), lambda i: (0, 0)),
            pl.BlockSpec((D, D), lambda i: (0, 0)),
            pl.BlockSpec((1, D), lambda i: (0, 0)),
            pl.BlockSpec((1, D), lambda i: (0, 0)),
            pl.BlockSpec((1, D), lambda i: (0, 0)),
            pl.BlockSpec((D, 2 * D), lambda i: (0, 0)),
            pl.BlockSpec((1, 2 * D), lambda i: (0, 0)),
            pl.BlockSpec((2 * D, D), lambda i: (0, 0)),
            pl.BlockSpec((1, D), lambda i: (0, 0)),
        ],
        out_specs=pl.BlockSpec((BN, D), lambda i: (i, 0)),
        out_shape=jax.ShapeDtypeStruct((NP, D), f32),
    )(xp, h, agg_part, deg_part, W_self, row1(b_self), W_agg, row1(b_agg),
      row1(ln2_g), row1(ln2_b), W_ffn1, row1(b_ffn1), W_ffn2, row1(b_ffn2))

    return (out[:N], edge_repr[:E])
